# baseline (speedup 1.0000x reference)
import functools

import jax
import jax.numpy as jnp
from jax.experimental import pallas as pl
from jax.experimental.pallas import tpu as pltpu

_LANES = 128


def _row_iota(rows):
    return jax.lax.broadcasted_iota(jnp.int32, (rows, 1), 0)


def _lane_iota():
    return jax.lax.broadcasted_iota(jnp.int32, (1, _LANES), 1)


def _cmpx_row(v, t, keep_min):
    l, rows, _ = v.shape
    v5 = v.reshape(l, rows // (2 * t), 2, t, _LANES)
    part = jnp.concatenate([v5[:, :, 1:2], v5[:, :, 0:1]], axis=2)
    part = part.reshape(l, rows, _LANES)
    mn = jnp.minimum(v, part)
    mx = jnp.maximum(v, part)
    return jnp.where(keep_min, mn, mx)


def _cmpx_row_fine(v, t, keep_min):
    l, rows, _ = v.shape
    v4 = v.reshape(l, rows // 8, 8, _LANES)
    down = jnp.roll(v4, -t, axis=2).reshape(l, rows, _LANES)
    up = jnp.roll(v4, t, axis=2).reshape(l, rows, _LANES)
    bj = (_row_iota(rows) // t) & 1
    part = jnp.where((bj == 0)[None], down, up)
    mn = jnp.minimum(v, part)
    mx = jnp.maximum(v, part)
    return jnp.where(keep_min, mn, mx)


def _cmpx(v, t, keep_min):
    if t >= 8:
        return _cmpx_row(v, t, keep_min)
    return _cmpx_row_fine(v, t, keep_min)


def _block_swap(v):
    l, rows, _ = v.shape
    v4 = v.reshape(l, rows // 128, 128, _LANES)
    v4 = jnp.swapaxes(v4, 2, 3)
    return v4.reshape(l, rows, _LANES)


def _g_bit(b, natural, rowi, lanei):
    if natural:
        return ((rowi >> (b - 7)) & 1) if b >= 7 else ((lanei >> b) & 1)
    return ((lanei >> (b - 7)) & 1) if b >= 7 else ((rowi >> b) & 1)


def _sort_chunks(v):
    desc = (jax.lax.broadcasted_iota(jnp.int32, (6, 1, 1), 0) % 3) != 0
    rowi = _row_iota(128)
    lanei = _lane_iota()
    natural = True
    for k in range(1, 15):
        for j in range(k - 1, -1, -1):
            if (j >= 7) != natural:
                v = _block_swap(v)
                natural = not natural
            bj = _g_bit(j, natural, rowi, lanei)
            bk = _g_bit(k, natural, rowi, lanei) if k < 14 else 0
            keep = (bj == bk)[None] != desc
            v = _cmpx(v, 1 << (j - 7 if natural else j), keep)
    if not natural:
        v = _block_swap(v)
    return v


def _merge_asc(v, log2n):
    l, rows, _ = v.shape
    rowi = _row_iota(rows)
    for j in range(log2n - 1, 6, -1):
        keep = (((rowi >> (j - 7)) & 1) == 0)[None]
        v = _cmpx(v, 1 << (j - 7), keep)
    v = _block_swap(v)
    for j in range(6, -1, -1):
        keep = (((rowi >> j) & 1) == 0)[None]
        v = _cmpx(v, 1 << j, keep)
    return _block_swap(v)


def _emd_body(x_ref, y_ref, o_ref, *, nreal):
    v = jnp.concatenate([x_ref[...], y_ref[...]], axis=0)
    v = v.reshape(2, 3, 128, _LANES).reshape(6, 128, _LANES)
    v = _sort_chunks(v)
    v = v.reshape(2, 3, 128, _LANES)
    m = v[:, 0:2].reshape(2, 256, _LANES)
    m = _merge_asc(m, 15)
    infs = jnp.full((2, 1, 128, _LANES), jnp.inf, jnp.float32)
    w = jnp.concatenate(
        [m.reshape(2, 2, 128, _LANES), infs, v[:, 2:3]], axis=1
    ).reshape(2, 512, _LANES)
    w = _merge_asc(w, 16)
    d = w[0, :384] - w[1, :384]
    o_ref[0] = jnp.full((8, _LANES), jnp.sum(d * d), jnp.float32)


def _emd_call(xp, yp, nreal):
    b = xp.shape[0]
    body = functools.partial(_emd_body, nreal=nreal)
    return pl.pallas_call(
        body,
        grid=(b,),
        in_specs=[
            pl.BlockSpec((1, 384, _LANES), lambda i: (i, 0, 0)),
            pl.BlockSpec((1, 384, _LANES), lambda i: (i, 0, 0)),
        ],
        out_specs=pl.BlockSpec((1, 8, _LANES), lambda i: (i, 0, 0)),
        out_shape=jax.ShapeDtypeStruct((b, 8, _LANES), jnp.float32),
        compiler_params=pltpu.CompilerParams(
            dimension_semantics=("parallel",)
        ),
    )(xp, yp)


def kernel(x, y):
    b = x.shape[0]
    n = x.shape[1] * x.shape[2]
    xp = jnp.swapaxes(x, 1, 2).reshape(b, 384, _LANES)
    yp = jnp.swapaxes(y, 1, 2).reshape(b, 384, _LANES)
    out = _emd_call(xp, yp, n)
    return jnp.sum(out[:, 0, 0]) / (b * n)


# speedup vs baseline: 1.0167x; 1.0167x over previous
import functools

import jax
import jax.numpy as jnp
from jax.experimental import pallas as pl
from jax.experimental.pallas import tpu as pltpu

_LANES = 128


def _row_iota(rows):
    return jax.lax.broadcasted_iota(jnp.int32, (rows, 1), 0)


def _lane_iota():
    return jax.lax.broadcasted_iota(jnp.int32, (1, _LANES), 1)


def _cmpx_row(v, t, keep_min):
    l, rows, _ = v.shape
    v5 = v.reshape(l, rows // (2 * t), 2, t, _LANES)
    part = jnp.concatenate([v5[:, :, 1:2], v5[:, :, 0:1]], axis=2)
    part = part.reshape(l, rows, _LANES)
    mn = jnp.minimum(v, part)
    mx = jnp.maximum(v, part)
    return jnp.where(keep_min, mn, mx)


def _cmpx_row_fine(v, t, keep_min):
    l, rows, _ = v.shape
    v4 = v.reshape(l, rows // 8, 8, _LANES)
    down = jnp.roll(v4, -t, axis=2).reshape(l, rows, _LANES)
    up = jnp.roll(v4, t, axis=2).reshape(l, rows, _LANES)
    bj = (_row_iota(rows) // t) & 1
    part = jnp.where((bj == 0)[None], down, up)
    mn = jnp.minimum(v, part)
    mx = jnp.maximum(v, part)
    return jnp.where(keep_min, mn, mx)


def _cmpx(v, t, keep_min):
    if t >= 8:
        return _cmpx_row(v, t, keep_min)
    return _cmpx_row_fine(v, t, keep_min)


def _block_swap(v):
    l, rows, _ = v.shape
    v4 = v.reshape(l, rows // 128, 128, _LANES)
    v4 = jnp.swapaxes(v4, 2, 3)
    return v4.reshape(l, rows, _LANES)


def _g_bit(b, natural, rowi, lanei):
    if natural:
        return ((rowi >> (b - 7)) & 1) if b >= 7 else ((lanei >> b) & 1)
    return ((lanei >> (b - 7)) & 1) if b >= 7 else ((rowi >> b) & 1)


def _sort_chunks(v):
    desc = ((jax.lax.broadcasted_iota(jnp.int32, (6, 1, 1), 0) % 3) != 0)
    desc = desc.astype(jnp.int32)
    rowi = _row_iota(128)
    lanei = _lane_iota()
    natural = True
    for k in range(1, 15):
        for j in range(k - 1, -1, -1):
            if (j >= 7) != natural:
                v = _block_swap(v)
                natural = not natural
            bj = _g_bit(j, natural, rowi, lanei)
            bk = _g_bit(k, natural, rowi, lanei) if k < 14 else 0
            keep = bj[None] == (bk ^ desc)
            v = _cmpx(v, 1 << (j - 7 if natural else j), keep)
    if not natural:
        v = _block_swap(v)
    return v


def _merge_asc(v, log2n):
    l, rows, _ = v.shape
    rowi = _row_iota(rows)
    for j in range(log2n - 1, 6, -1):
        keep = (((rowi >> (j - 7)) & 1) == 0)[None]
        v = _cmpx(v, 1 << (j - 7), keep)
    v = _block_swap(v)
    for j in range(6, -1, -1):
        keep = (((rowi >> j) & 1) == 0)[None]
        v = _cmpx(v, 1 << j, keep)
    return _block_swap(v)


def _emd_body(x_ref, y_ref, o_ref, *, nreal):
    v = jnp.concatenate([x_ref[...], y_ref[...]], axis=0)
    v = v.reshape(2, 3, 128, _LANES).reshape(6, 128, _LANES)
    v = _sort_chunks(v)
    v = v.reshape(2, 3, 128, _LANES)
    m = v[:, 0:2].reshape(2, 256, _LANES)
    m = _merge_asc(m, 15)
    infs = jnp.full((2, 1, 128, _LANES), jnp.inf, jnp.float32)
    w = jnp.concatenate(
        [m.reshape(2, 2, 128, _LANES), infs, v[:, 2:3]], axis=1
    ).reshape(2, 512, _LANES)
    w = _merge_asc(w, 16)
    d = w[0, :384] - w[1, :384]
    o_ref[0] = jnp.full((8, _LANES), jnp.sum(d * d), jnp.float32)


def _emd_call(xp, yp, nreal):
    b = xp.shape[0]
    body = functools.partial(_emd_body, nreal=nreal)
    return pl.pallas_call(
        body,
        grid=(b,),
        in_specs=[
            pl.BlockSpec((1, 384, _LANES), lambda i: (i, 0, 0)),
            pl.BlockSpec((1, 384, _LANES), lambda i: (i, 0, 0)),
        ],
        out_specs=pl.BlockSpec((1, 8, _LANES), lambda i: (i, 0, 0)),
        out_shape=jax.ShapeDtypeStruct((b, 8, _LANES), jnp.float32),
        compiler_params=pltpu.CompilerParams(
            dimension_semantics=("parallel",)
        ),
    )(xp, yp)


def kernel(x, y):
    b = x.shape[0]
    n = x.shape[1] * x.shape[2]
    xp = jnp.swapaxes(x, 1, 2).reshape(b, 384, _LANES)
    yp = jnp.swapaxes(y, 1, 2).reshape(b, 384, _LANES)
    out = _emd_call(xp, yp, n)
    return jnp.sum(out[:, 0, 0]) / (b * n)
